# baseline (speedup 1.0000x reference)
import functools

import jax
import jax.numpy as jnp
from jax import lax
from jax.experimental import pallas as pl
from jax.experimental.pallas import tpu as pltpu
from jax.experimental.pallas import tpu_sc as plsc

B = 512
NVAR = 6
M = 7
R = 16384
OUT_M = 9
NC = 2
NS = 16
NW = NC * NS
L = 16
NPAIR = 3
PW = M * M
CH = 2048
NCH = R // CH
NOUT2 = 2 * OUT_M


def _bf16_round(v):
    y = lax.bitcast_convert_type(v, jnp.int32)
    odd = lax.shift_right_logical(y, 16) & jnp.int32(1)
    r = (y + jnp.int32(0x7FFF) + odd) & jnp.int32(-65536)
    return lax.bitcast_convert_type(r, jnp.float32)


def _sc_body(xw_hbm, csp_hbm, rules_hbm, oc_hbm, out_hbm,
             xv_ref, csv_ref, oc_ref, pt0_ref, pt1_ref, pt2_ref,
             rbuf_ref, code_ref, obase_ref, s_ref, outb_ref):
    wid = lax.axis_index("s") * NC + lax.axis_index("c")
    iota = lax.iota(jnp.int32, L)

    pltpu.sync_copy(xw_hbm.at[pl.ds(wid * (NVAR * L), NVAR * L)], xv_ref)
    pltpu.sync_copy(csp_hbm, csv_ref)
    pltpu.sync_copy(oc_hbm, oc_ref)

    ocr0 = _bf16_round(oc_ref[pl.ds(0, L)])
    ocr1 = _bf16_round(oc_ref[pl.ds(L, L)])

    pts = (pt0_ref, pt1_ref, pt2_ref)
    for a in range(NPAIR):
        va, vb = 2 * a, 2 * a + 1
        xa = xv_ref[pl.ds(va * L, L)]
        xb = xv_ref[pl.ds(vb * L, L)]
        ca_row = csv_ref[pl.ds(va * L, L)]
        cb_row = csv_ref[pl.ds(vb * L, L)]
        sa_row = csv_ref[pl.ds((NVAR + va) * L, L)]
        sb_row = csv_ref[pl.ds((NVAR + vb) * L, L)]
        fa, fb = [], []
        for m in range(M):
            da = xa - ca_row[m]
            ka = 0.5 / (jnp.full((L,), sa_row[m]) * jnp.full((L,), sa_row[m]))
            fa.append(jnp.exp(-(da * da) * ka))
            db = xb - cb_row[m]
            kb = 0.5 / (jnp.full((L,), sb_row[m]) * jnp.full((L,), sb_row[m]))
            fb.append(jnp.exp(-(db * db) * kb))
        for i in range(M):
            for j in range(M):
                pts[a][pl.ds((M * i + j) * L, L)] = jnp.minimum(fa[i], fb[j])

    for jj in range(NOUT2):
        s_ref[pl.ds(jj * L, L)] = jnp.zeros((L,), jnp.float32)

    for ch in range(NCH):
        pltpu.sync_copy(rules_hbm.at[pl.ds(ch * 8 * CH, 8 * CH)], rbuf_ref)

        def _codes(i, carry, ch=ch):
            def fld(f):
                return rbuf_ref[pl.ds(f * CH + i * L, L)]
            g = ch * CH + i * L
            code_ref[pl.ds(0 * R + g, L)] = (
                fld(0) * M + (fld(1) - M)) * L
            code_ref[pl.ds(1 * R + g, L)] = (
                (fld(2) - 2 * M) * M + (fld(3) - 3 * M)) * L
            code_ref[pl.ds(2 * R + g, L)] = (
                (fld(4) - 4 * M) * M + (fld(5) - 5 * M)) * L
            obase_ref[pl.ds(0 * R + g, L)] = fld(6) * L
            obase_ref[pl.ds(1 * R + g, L)] = fld(7) * L
            return carry

        lax.fori_loop(0, CH // L, _codes, 0, unroll=False)

    def _lane(v, t):
        return jnp.take(v, jnp.full((L,), t, jnp.int32))

    def _gather_w(c0v, c1v, c2v, t):
        w0 = plsc.load_gather(pt0_ref, [_lane(c0v, t) + iota])
        w1 = plsc.load_gather(pt1_ref, [_lane(c1v, t) + iota])
        w2 = plsc.load_gather(pt2_ref, [_lane(c2v, t) + iota])
        return jnp.minimum(jnp.minimum(w0, w1), w2)

    def _norm_block(k, accs):
        c0v = code_ref[pl.ds(0 * R + k * L, L)]
        c1v = code_ref[pl.ds(1 * R + k * L, L)]
        c2v = code_ref[pl.ds(2 * R + k * L, L)]
        accs = list(accs)
        for t in range(L):
            accs[t % 4] = accs[t % 4] + _gather_w(c0v, c1v, c2v, t)
        return tuple(accs)

    z = jnp.zeros((L,), jnp.float32)
    a0, a1, a2, a3 = lax.fori_loop(0, R // L, _norm_block, (z, z, z, z),
                                   unroll=2)
    norm = (a0 + a1) + (a2 + a3)

    inv = 1.0 / jnp.maximum(norm, 1e-12)

    for c in range(PW):
        pt0_ref[pl.ds(c * L, L)] = _bf16_round(pt0_ref[pl.ds(c * L, L)] * inv)
        pt1_ref[pl.ds(c * L, L)] = _bf16_round(pt1_ref[pl.ds(c * L, L)] * inv)
        pt2_ref[pl.ds(c * L, L)] = _bf16_round(pt2_ref[pl.ds(c * L, L)] * inv)

    def _accum_block(k, carry):
        c0v = code_ref[pl.ds(0 * R + k * L, L)]
        c1v = code_ref[pl.ds(1 * R + k * L, L)]
        c2v = code_ref[pl.ds(2 * R + k * L, L)]
        o0v = obase_ref[pl.ds(0 * R + k * L, L)]
        o1v = obase_ref[pl.ds(1 * R + k * L, L)]
        for t in range(L):
            nw = _gather_w(c0v, c1v, c2v, t)
            plsc.addupdate_scatter(s_ref, [iota + _lane(o0v, t)], nw)
            plsc.addupdate_scatter(s_ref, [iota + _lane(o1v, t)], nw)
        return carry

    lax.fori_loop(0, R // L, _accum_block, 0, unroll=2)

    acc0 = jnp.zeros((L,), jnp.float32)
    for jj in range(OUT_M):
        acc0 = acc0 + s_ref[pl.ds(jj * L, L)] * ocr0[jj]
    acc1 = jnp.zeros((L,), jnp.float32)
    for jj in range(OUT_M, NOUT2):
        scal = ocr0[jj] if jj < L else ocr1[jj - L]
        acc1 = acc1 + s_ref[pl.ds(jj * L, L)] * scal
    outb_ref[pl.ds(0, L)] = acc0
    outb_ref[pl.ds(L, L)] = acc1
    pltpu.sync_copy(outb_ref, out_hbm.at[pl.ds(wid * 2 * L, 2 * L)])


@jax.jit
def kernel(x, centers, sigmas, out_centers, input_rules, output_rules):
    xw = x.T.reshape(NVAR, NW, L).transpose(1, 0, 2).reshape(-1)
    csp = jnp.pad(jnp.concatenate([centers, sigmas], axis=0),
                  ((0, 0), (0, L - M)),
                  constant_values=1.0).reshape(-1)
    rules8 = jnp.concatenate([input_rules.T, output_rules.T],
                             axis=0).astype(jnp.int32)
    rulesf = rules8.reshape(8, NCH, CH).transpose(1, 0, 2).reshape(-1)
    ocp = jnp.pad(out_centers, (0, 32 - NOUT2))

    mesh = plsc.VectorSubcoreMesh(core_axis_name="c", subcore_axis_name="s",
                                  num_cores=NC, num_subcores=NS)
    run = functools.partial(
        pl.kernel,
        out_type=jax.ShapeDtypeStruct((NW * 2 * L,), jnp.float32),
        mesh=mesh,
        scratch_types=[
            pltpu.VMEM((NVAR * L,), jnp.float32),
            pltpu.VMEM((2 * NVAR * L,), jnp.float32),
            pltpu.VMEM((32,), jnp.float32),
            pltpu.VMEM((PW * L + L,), jnp.float32),
            pltpu.VMEM((PW * L + L,), jnp.float32),
            pltpu.VMEM((PW * L + L,), jnp.float32),
            pltpu.VMEM((8 * CH,), jnp.int32),
            pltpu.VMEM((NPAIR * R,), jnp.int32),
            pltpu.VMEM((2 * R,), jnp.int32),
            pltpu.VMEM((NOUT2 * L,), jnp.float32),
            pltpu.VMEM((2 * L,), jnp.float32),
        ],
        compiler_params=pltpu.CompilerParams(needs_layout_passes=False),
    )(_sc_body)
    flat = run(xw, csp, rulesf, ocp)
    return flat.reshape(NW, 2, L).transpose(0, 2, 1).reshape(B, 2)


# speedup vs baseline: 1.0615x; 1.0615x over previous
import functools

import jax
import jax.numpy as jnp
from jax import lax
from jax.experimental import pallas as pl
from jax.experimental.pallas import tpu as pltpu
from jax.experimental.pallas import tpu_sc as plsc

B = 512
NVAR = 6
M = 7
R = 16384
OUT_M = 9
NC = 2
NS = 16
NW = NC * NS
L = 16
NPAIR = 3
PW = M * M
CH = 2048
NCH = R // CH
NOUT2 = 2 * OUT_M


def _bf16_round(v):
    y = lax.bitcast_convert_type(v, jnp.int32)
    odd = lax.shift_right_logical(y, 16) & jnp.int32(1)
    r = (y + jnp.int32(0x7FFF) + odd) & jnp.int32(-65536)
    return lax.bitcast_convert_type(r, jnp.float32)


def _sc_body(xw_hbm, csp_hbm, rules_hbm, oc_hbm, out_hbm,
             xv_ref, csv_ref, oc_ref, pt0_ref, pt1_ref, pt2_ref,
             rbuf_ref, rbuf2_ref, code_ref, obase_ref, s_ref, outb_ref,
             sem0, sem1):
    wid = lax.axis_index("s") * NC + lax.axis_index("c")
    iota = lax.iota(jnp.int32, L)

    pltpu.sync_copy(xw_hbm.at[pl.ds(wid * (NVAR * L), NVAR * L)], xv_ref)
    pltpu.sync_copy(csp_hbm, csv_ref)
    pltpu.sync_copy(oc_hbm, oc_ref)

    ocr0 = _bf16_round(oc_ref[pl.ds(0, L)])
    ocr1 = _bf16_round(oc_ref[pl.ds(L, L)])

    pts = (pt0_ref, pt1_ref, pt2_ref)
    for a in range(NPAIR):
        va, vb = 2 * a, 2 * a + 1
        xa = xv_ref[pl.ds(va * L, L)]
        xb = xv_ref[pl.ds(vb * L, L)]
        ca_row = csv_ref[pl.ds(va * L, L)]
        cb_row = csv_ref[pl.ds(vb * L, L)]
        sa_row = csv_ref[pl.ds((NVAR + va) * L, L)]
        sb_row = csv_ref[pl.ds((NVAR + vb) * L, L)]
        fa, fb = [], []
        for m in range(M):
            da = xa - ca_row[m]
            ka = 0.5 / (jnp.full((L,), sa_row[m]) * jnp.full((L,), sa_row[m]))
            fa.append(jnp.exp(-(da * da) * ka))
            db = xb - cb_row[m]
            kb = 0.5 / (jnp.full((L,), sb_row[m]) * jnp.full((L,), sb_row[m]))
            fb.append(jnp.exp(-(db * db) * kb))
        for i in range(M):
            for j in range(M):
                pts[a][pl.ds((M * i + j) * L, L)] = jnp.minimum(fa[i], fb[j])

    for jj in range(NOUT2):
        s_ref[pl.ds(jj * L, L)] = jnp.zeros((L,), jnp.float32)

    rbufs = (rbuf_ref, rbuf2_ref)
    cps = [None, None]
    cps[0] = pltpu.async_copy(rules_hbm.at[pl.ds(0, 8 * CH)], rbufs[0],
                              sem0)
    sems = (sem0, sem1)
    for ch in range(NCH):
        nxt = ch + 1
        if nxt < NCH:
            cps[nxt % 2] = pltpu.async_copy(
                rules_hbm.at[pl.ds(nxt * 8 * CH, 8 * CH)], rbufs[nxt % 2],
                sems[nxt % 2])
        cps[ch % 2].wait()
        rb = rbufs[ch % 2]

        def _codes(i, carry, rb=rb, ch=ch):
            def fld(f):
                return rb[pl.ds(f * CH + i * L, L)]
            g = ch * CH + i * L
            c0 = (fld(0) * M + (fld(1) - M)) * L
            c1 = ((fld(2) - 2 * M) * M + (fld(3) - 3 * M)) * L
            c2 = ((fld(4) - 4 * M) * M + (fld(5) - 5 * M)) * L
            code_ref[pl.ds(g, L)] = (
                c0 | lax.shift_left(c1, 10) | lax.shift_left(c2, 20))
            obase_ref[pl.ds(g, L)] = (
                fld(6) * L | lax.shift_left(fld(7) * L, 9))
            return carry

        lax.fori_loop(0, CH // L, _codes, 0, unroll=False)

    def _lane(v, t):
        return jnp.take(v, jnp.full((L,), t, jnp.int32))

    m10 = jnp.int32(1023)

    def _gather_w(cv, t):
        b = _lane(cv, t)
        w0 = plsc.load_gather(pt0_ref, [(b & m10) + iota])
        w1 = plsc.load_gather(pt1_ref,
                              [(lax.shift_right_logical(b, 10) & m10) + iota])
        w2 = plsc.load_gather(pt2_ref,
                              [lax.shift_right_logical(b, 20) + iota])
        return jnp.minimum(jnp.minimum(w0, w1), w2)

    def _norm_block(k, accs):
        cv = code_ref[pl.ds(k * L, L)]
        accs = list(accs)
        for t in range(L):
            accs[t % 4] = accs[t % 4] + _gather_w(cv, t)
        return tuple(accs)

    z = jnp.zeros((L,), jnp.float32)
    a0, a1, a2, a3 = lax.fori_loop(0, R // L, _norm_block, (z, z, z, z),
                                   unroll=2)
    norm = (a0 + a1) + (a2 + a3)

    inv = 1.0 / jnp.maximum(norm, 1e-12)

    for c in range(PW):
        pt0_ref[pl.ds(c * L, L)] = _bf16_round(pt0_ref[pl.ds(c * L, L)] * inv)
        pt1_ref[pl.ds(c * L, L)] = _bf16_round(pt1_ref[pl.ds(c * L, L)] * inv)
        pt2_ref[pl.ds(c * L, L)] = _bf16_round(pt2_ref[pl.ds(c * L, L)] * inv)

    m9 = jnp.int32(511)

    def _accum_block(k, carry):
        cv = code_ref[pl.ds(k * L, L)]
        ov = obase_ref[pl.ds(k * L, L)]
        for t in range(L):
            nw = _gather_w(cv, t)
            ob = _lane(ov, t)
            plsc.addupdate_scatter(s_ref, [iota + (ob & m9)], nw)
            plsc.addupdate_scatter(
                s_ref, [iota + lax.shift_right_logical(ob, 9)], nw)
        return carry

    lax.fori_loop(0, R // L, _accum_block, 0, unroll=2)

    acc0 = jnp.zeros((L,), jnp.float32)
    for jj in range(OUT_M):
        acc0 = acc0 + s_ref[pl.ds(jj * L, L)] * ocr0[jj]
    acc1 = jnp.zeros((L,), jnp.float32)
    for jj in range(OUT_M, NOUT2):
        scal = ocr0[jj] if jj < L else ocr1[jj - L]
        acc1 = acc1 + s_ref[pl.ds(jj * L, L)] * scal
    outb_ref[pl.ds(0, L)] = acc0
    outb_ref[pl.ds(L, L)] = acc1
    pltpu.sync_copy(outb_ref, out_hbm.at[pl.ds(wid * 2 * L, 2 * L)])


@jax.jit
def kernel(x, centers, sigmas, out_centers, input_rules, output_rules):
    xw = x.T.reshape(NVAR, NW, L).transpose(1, 0, 2).reshape(-1)
    csp = jnp.pad(jnp.concatenate([centers, sigmas], axis=0),
                  ((0, 0), (0, L - M)),
                  constant_values=1.0).reshape(-1)
    rules8 = jnp.concatenate([input_rules.T, output_rules.T],
                             axis=0).astype(jnp.int32)
    rulesf = rules8.reshape(8, NCH, CH).transpose(1, 0, 2).reshape(-1)
    ocp = jnp.pad(out_centers, (0, 32 - NOUT2))

    mesh = plsc.VectorSubcoreMesh(core_axis_name="c", subcore_axis_name="s",
                                  num_cores=NC, num_subcores=NS)
    run = functools.partial(
        pl.kernel,
        out_type=jax.ShapeDtypeStruct((NW * 2 * L,), jnp.float32),
        mesh=mesh,
        scratch_types=[
            pltpu.VMEM((NVAR * L,), jnp.float32),
            pltpu.VMEM((2 * NVAR * L,), jnp.float32),
            pltpu.VMEM((32,), jnp.float32),
            pltpu.VMEM((PW * L + L,), jnp.float32),
            pltpu.VMEM((PW * L + L,), jnp.float32),
            pltpu.VMEM((PW * L + L,), jnp.float32),
            pltpu.VMEM((8 * CH,), jnp.int32),
            pltpu.VMEM((8 * CH,), jnp.int32),
            pltpu.VMEM((R,), jnp.int32),
            pltpu.VMEM((R,), jnp.int32),
            pltpu.VMEM((NOUT2 * L,), jnp.float32),
            pltpu.VMEM((2 * L,), jnp.float32),
            pltpu.SemaphoreType.DMA,
            pltpu.SemaphoreType.DMA,
        ],
        compiler_params=pltpu.CompilerParams(needs_layout_passes=False),
    )(_sc_body)
    flat = run(xw, csp, rulesf, ocp)
    return flat.reshape(NW, 2, L).transpose(0, 2, 1).reshape(B, 2)


# speedup vs baseline: 1.2779x; 1.2039x over previous
import functools

import jax
import jax.numpy as jnp
from jax import lax
from jax.experimental import pallas as pl
from jax.experimental.pallas import tpu as pltpu
from jax.experimental.pallas import tpu_sc as plsc

B = 512
NVAR = 6
M = 7
R = 16384
OUT_M = 9
NC = 2
NS = 16
NW = NC * NS
L = 16
NPAIR = 3
PW = M * M
CH = 2048
NCH = R // CH
NOUT2 = 2 * OUT_M


def _bf16_round(v):
    y = lax.bitcast_convert_type(v, jnp.int32)
    odd = lax.shift_right_logical(y, 16) & jnp.int32(1)
    r = (y + jnp.int32(0x7FFF) + odd) & jnp.int32(-65536)
    return lax.bitcast_convert_type(r, jnp.float32)


def _sc_body(xw_hbm, csp_hbm, rules_hbm, oc_hbm, out_hbm,
             xv_ref, csv_ref, oc_ref, pt0_ref, pt1_ref, pt2_ref,
             rbuf_ref, rbuf2_ref, code_ref, obase_ref, s_ref, outb_ref,
             sem0, sem1):
    wid = lax.axis_index("s") * NC + lax.axis_index("c")
    iota = lax.iota(jnp.int32, L)

    pltpu.sync_copy(xw_hbm.at[pl.ds(wid * (NVAR * L), NVAR * L)], xv_ref)
    pltpu.sync_copy(csp_hbm, csv_ref)
    pltpu.sync_copy(oc_hbm, oc_ref)

    ocr0 = _bf16_round(oc_ref[pl.ds(0, L)])
    ocr1 = _bf16_round(oc_ref[pl.ds(L, L)])

    pts = (pt0_ref, pt1_ref, pt2_ref)
    for a in range(NPAIR):
        va, vb = 2 * a, 2 * a + 1
        xa = xv_ref[pl.ds(va * L, L)]
        xb = xv_ref[pl.ds(vb * L, L)]
        ca_row = csv_ref[pl.ds(va * L, L)]
        cb_row = csv_ref[pl.ds(vb * L, L)]
        sa_row = csv_ref[pl.ds((NVAR + va) * L, L)]
        sb_row = csv_ref[pl.ds((NVAR + vb) * L, L)]
        fa, fb = [], []
        for m in range(M):
            da = xa - ca_row[m]
            ka = 0.5 / (jnp.full((L,), sa_row[m]) * jnp.full((L,), sa_row[m]))
            fa.append(jnp.exp(-(da * da) * ka))
            db = xb - cb_row[m]
            kb = 0.5 / (jnp.full((L,), sb_row[m]) * jnp.full((L,), sb_row[m]))
            fb.append(jnp.exp(-(db * db) * kb))
        for i in range(M):
            for j in range(M):
                pts[a][pl.ds((M * i + j) * L, L)] = jnp.minimum(fa[i], fb[j])

    for jj in range(NOUT2):
        s_ref[pl.ds(jj * L, L)] = jnp.zeros((L,), jnp.float32)

    rbufs = (rbuf_ref, rbuf2_ref)
    cps = [None, None]
    cps[0] = pltpu.async_copy(rules_hbm.at[pl.ds(0, 8 * CH)], rbufs[0],
                              sem0)
    sems = (sem0, sem1)
    for ch in range(NCH):
        nxt = ch + 1
        if nxt < NCH:
            cps[nxt % 2] = pltpu.async_copy(
                rules_hbm.at[pl.ds(nxt * 8 * CH, 8 * CH)], rbufs[nxt % 2],
                sems[nxt % 2])
        cps[ch % 2].wait()
        rb = rbufs[ch % 2]

        def _codes(i, carry, rb=rb, ch=ch):
            def fld(f):
                return rb[pl.ds(f * CH + i * L, L)]
            g = ch * CH + i * L
            c0 = (fld(0) * M + (fld(1) - M)) * L
            c1 = ((fld(2) - 2 * M) * M + (fld(3) - 3 * M)) * L
            c2 = ((fld(4) - 4 * M) * M + (fld(5) - 5 * M)) * L
            code_ref[pl.ds(g, L)] = (
                c0 | lax.shift_left(c1, 10) | lax.shift_left(c2, 20))
            obase_ref[pl.ds(g, L)] = (
                fld(6) * L | lax.shift_left(fld(7) * L, 9))
            return carry

        lax.fori_loop(0, CH // L, _codes, 0, unroll=False)

    def _lane(v, t):
        return jnp.take(v, jnp.full((L,), t, jnp.int32))

    m10 = jnp.int32(1023)

    def _gather_w(cv, t):
        b = cv[t]
        w0 = pt0_ref[pl.ds(b & m10, L)]
        w1 = pt1_ref[pl.ds(lax.shift_right_logical(b, 10) & m10, L)]
        w2 = pt2_ref[pl.ds(lax.shift_right_logical(b, 20), L)]
        return jnp.minimum(jnp.minimum(w0, w1), w2)

    def _norm_block(k, accs):
        cv = code_ref[pl.ds(k * L, L)]
        accs = list(accs)
        for t in range(L):
            accs[t % 4] = accs[t % 4] + _gather_w(cv, t)
        return tuple(accs)

    z = jnp.zeros((L,), jnp.float32)
    a0, a1, a2, a3 = lax.fori_loop(0, R // L, _norm_block, (z, z, z, z),
                                   unroll=2)
    norm = (a0 + a1) + (a2 + a3)

    inv = 1.0 / jnp.maximum(norm, 1e-12)

    for c in range(PW):
        pt0_ref[pl.ds(c * L, L)] = _bf16_round(pt0_ref[pl.ds(c * L, L)] * inv)
        pt1_ref[pl.ds(c * L, L)] = _bf16_round(pt1_ref[pl.ds(c * L, L)] * inv)
        pt2_ref[pl.ds(c * L, L)] = _bf16_round(pt2_ref[pl.ds(c * L, L)] * inv)

    m9 = jnp.int32(511)

    def _accum_block(k, carry):
        cv = code_ref[pl.ds(k * L, L)]
        ov = obase_ref[pl.ds(k * L, L)]
        for t in range(L):
            nw = _gather_w(cv, t)
            ob = ov[t]
            plsc.addupdate_scatter(s_ref, [iota + (ob & m9)], nw)
            plsc.addupdate_scatter(
                s_ref, [iota + lax.shift_right_logical(ob, 9)], nw)
        return carry

    lax.fori_loop(0, R // L, _accum_block, 0, unroll=2)

    acc0 = jnp.zeros((L,), jnp.float32)
    for jj in range(OUT_M):
        acc0 = acc0 + s_ref[pl.ds(jj * L, L)] * ocr0[jj]
    acc1 = jnp.zeros((L,), jnp.float32)
    for jj in range(OUT_M, NOUT2):
        scal = ocr0[jj] if jj < L else ocr1[jj - L]
        acc1 = acc1 + s_ref[pl.ds(jj * L, L)] * scal
    outb_ref[pl.ds(0, L)] = acc0
    outb_ref[pl.ds(L, L)] = acc1
    pltpu.sync_copy(outb_ref, out_hbm.at[pl.ds(wid * 2 * L, 2 * L)])


@jax.jit
def kernel(x, centers, sigmas, out_centers, input_rules, output_rules):
    xw = x.T.reshape(NVAR, NW, L).transpose(1, 0, 2).reshape(-1)
    csp = jnp.pad(jnp.concatenate([centers, sigmas], axis=0),
                  ((0, 0), (0, L - M)),
                  constant_values=1.0).reshape(-1)
    rules8 = jnp.concatenate([input_rules.T, output_rules.T],
                             axis=0).astype(jnp.int32)
    rulesf = rules8.reshape(8, NCH, CH).transpose(1, 0, 2).reshape(-1)
    ocp = jnp.pad(out_centers, (0, 32 - NOUT2))

    mesh = plsc.VectorSubcoreMesh(core_axis_name="c", subcore_axis_name="s",
                                  num_cores=NC, num_subcores=NS)
    run = functools.partial(
        pl.kernel,
        out_type=jax.ShapeDtypeStruct((NW * 2 * L,), jnp.float32),
        mesh=mesh,
        scratch_types=[
            pltpu.VMEM((NVAR * L,), jnp.float32),
            pltpu.VMEM((2 * NVAR * L,), jnp.float32),
            pltpu.VMEM((32,), jnp.float32),
            pltpu.VMEM((PW * L + L,), jnp.float32),
            pltpu.VMEM((PW * L + L,), jnp.float32),
            pltpu.VMEM((PW * L + L,), jnp.float32),
            pltpu.VMEM((8 * CH,), jnp.int32),
            pltpu.VMEM((8 * CH,), jnp.int32),
            pltpu.VMEM((R,), jnp.int32),
            pltpu.VMEM((R,), jnp.int32),
            pltpu.VMEM((NOUT2 * L,), jnp.float32),
            pltpu.VMEM((2 * L,), jnp.float32),
            pltpu.SemaphoreType.DMA,
            pltpu.SemaphoreType.DMA,
        ],
        compiler_params=pltpu.CompilerParams(needs_layout_passes=False),
    )(_sc_body)
    flat = run(xw, csp, rulesf, ocp)
    return flat.reshape(NW, 2, L).transpose(0, 2, 1).reshape(B, 2)


# speedup vs baseline: 1.3858x; 1.0844x over previous
import functools

import jax
import jax.numpy as jnp
from jax import lax
from jax.experimental import pallas as pl
from jax.experimental.pallas import tpu as pltpu
from jax.experimental.pallas import tpu_sc as plsc

B = 512
NVAR = 6
M = 7
R = 16384
OUT_M = 9
NC = 2
NS = 16
NW = NC * NS
L = 16
NPAIR = 3
PW = M * M
TW = M * M * M
CH = 2048
NCH = R // CH
NOUT2 = 2 * OUT_M


def _bf16_round(v):
    y = lax.bitcast_convert_type(v, jnp.int32)
    odd = lax.shift_right_logical(y, 16) & jnp.int32(1)
    r = (y + jnp.int32(0x7FFF) + odd) & jnp.int32(-65536)
    return lax.bitcast_convert_type(r, jnp.float32)


def _sc_body(xw_hbm, csp_hbm, rules_hbm, oc_hbm, out_hbm,
             xv_ref, csv_ref, oc_ref, pt0_ref, pt1_ref,
             rbuf_ref, rbuf2_ref, code_ref, obase_ref, s_ref, outb_ref,
             sem0, sem1):
    wid = lax.axis_index("s") * NC + lax.axis_index("c")
    iota = lax.iota(jnp.int32, L)

    pltpu.sync_copy(xw_hbm.at[pl.ds(wid * (NVAR * L), NVAR * L)], xv_ref)
    pltpu.sync_copy(csp_hbm, csv_ref)
    pltpu.sync_copy(oc_hbm, oc_ref)

    ocr0 = _bf16_round(oc_ref[pl.ds(0, L)])
    ocr1 = _bf16_round(oc_ref[pl.ds(L, L)])

    def _fuzz(v):
        xm = xv_ref[pl.ds(v * L, L)]
        c_row = csv_ref[pl.ds(v * L, L)]
        s_row = csv_ref[pl.ds((NVAR + v) * L, L)]
        out = []
        for m in range(M):
            d = xm - c_row[m]
            kk = 0.5 / (jnp.full((L,), s_row[m]) * jnp.full((L,), s_row[m]))
            out.append(jnp.exp(-(d * d) * kk))
        return out

    pts = (pt0_ref, pt1_ref)
    for a in range(2):
        fa = _fuzz(3 * a)
        fb = _fuzz(3 * a + 1)
        fc = _fuzz(3 * a + 2)
        for i in range(M):
            for j in range(M):
                pij = jnp.minimum(fa[i], fb[j])
                for k in range(M):
                    pts[a][pl.ds(((i * M + j) * M + k) * L, L)] = (
                        jnp.minimum(pij, fc[k]))

    for jj in range(NOUT2):
        s_ref[pl.ds(jj * L, L)] = jnp.zeros((L,), jnp.float32)

    rbufs = (rbuf_ref, rbuf2_ref)
    cps = [None, None]
    cps[0] = pltpu.async_copy(rules_hbm.at[pl.ds(0, 8 * CH)], rbufs[0],
                              sem0)
    sems = (sem0, sem1)
    for ch in range(NCH):
        nxt = ch + 1
        if nxt < NCH:
            cps[nxt % 2] = pltpu.async_copy(
                rules_hbm.at[pl.ds(nxt * 8 * CH, 8 * CH)], rbufs[nxt % 2],
                sems[nxt % 2])
        cps[ch % 2].wait()
        rb = rbufs[ch % 2]

        def _codes(i, carry, rb=rb, ch=ch):
            def fld(f):
                return rb[pl.ds(f * CH + i * L, L)]
            g = ch * CH + i * L
            c0 = (((fld(0) * M + (fld(1) - M)) * M) + fld(2) - 2 * M) * L
            c1 = ((((fld(3) - 3 * M) * M + (fld(4) - 4 * M)) * M)
                  + fld(5) - 5 * M) * L
            code_ref[pl.ds(g, L)] = c0 | lax.shift_left(c1, 13)
            obase_ref[pl.ds(g, L)] = (
                fld(6) * L | lax.shift_left(fld(7) * L, 9))
            return carry

        lax.fori_loop(0, CH // L, _codes, 0, unroll=False)

    def _lane(v, t):
        return jnp.take(v, jnp.full((L,), t, jnp.int32))

    m13 = jnp.int32(8191)

    def _gather_w(cv, t):
        b = cv[t]
        w0 = pt0_ref[pl.ds(b & m13, L)]
        w1 = pt1_ref[pl.ds(lax.shift_right_logical(b, 13), L)]
        return jnp.minimum(w0, w1)

    def _norm_block(k, accs):
        cv = code_ref[pl.ds(k * L, L)]
        accs = list(accs)
        for t in range(L):
            accs[t % 4] = accs[t % 4] + _gather_w(cv, t)
        return tuple(accs)

    z = jnp.zeros((L,), jnp.float32)
    a0, a1, a2, a3 = lax.fori_loop(0, R // L, _norm_block, (z, z, z, z),
                                   unroll=2)
    norm = (a0 + a1) + (a2 + a3)

    inv = 1.0 / jnp.maximum(norm, 1e-12)

    def _scale_block(c, carry):
        sl = pl.ds(c * L, L)
        pt0_ref[sl] = _bf16_round(pt0_ref[sl] * inv)
        pt1_ref[sl] = _bf16_round(pt1_ref[sl] * inv)
        return carry

    lax.fori_loop(0, TW, _scale_block, 0, unroll=4)

    m9 = jnp.int32(511)

    def _accum_block(k, carry):
        cv = code_ref[pl.ds(k * L, L)]
        ov = obase_ref[pl.ds(k * L, L)]
        for t in range(L):
            nw = _gather_w(cv, t)
            ob = ov[t]
            plsc.addupdate_scatter(s_ref, [iota + (ob & m9)], nw)
            plsc.addupdate_scatter(
                s_ref, [iota + lax.shift_right_logical(ob, 9)], nw)
        return carry

    lax.fori_loop(0, R // L, _accum_block, 0, unroll=2)

    acc0 = jnp.zeros((L,), jnp.float32)
    for jj in range(OUT_M):
        acc0 = acc0 + s_ref[pl.ds(jj * L, L)] * ocr0[jj]
    acc1 = jnp.zeros((L,), jnp.float32)
    for jj in range(OUT_M, NOUT2):
        scal = ocr0[jj] if jj < L else ocr1[jj - L]
        acc1 = acc1 + s_ref[pl.ds(jj * L, L)] * scal
    outb_ref[pl.ds(0, L)] = acc0
    outb_ref[pl.ds(L, L)] = acc1
    pltpu.sync_copy(outb_ref, out_hbm.at[pl.ds(wid * 2 * L, 2 * L)])


@jax.jit
def kernel(x, centers, sigmas, out_centers, input_rules, output_rules):
    xw = x.T.reshape(NVAR, NW, L).transpose(1, 0, 2).reshape(-1)
    csp = jnp.pad(jnp.concatenate([centers, sigmas], axis=0),
                  ((0, 0), (0, L - M)),
                  constant_values=1.0).reshape(-1)
    rules8 = jnp.concatenate([input_rules.T, output_rules.T],
                             axis=0).astype(jnp.int32)
    rulesf = rules8.reshape(8, NCH, CH).transpose(1, 0, 2).reshape(-1)
    ocp = jnp.pad(out_centers, (0, 32 - NOUT2))

    mesh = plsc.VectorSubcoreMesh(core_axis_name="c", subcore_axis_name="s",
                                  num_cores=NC, num_subcores=NS)
    run = functools.partial(
        pl.kernel,
        out_type=jax.ShapeDtypeStruct((NW * 2 * L,), jnp.float32),
        mesh=mesh,
        scratch_types=[
            pltpu.VMEM((NVAR * L,), jnp.float32),
            pltpu.VMEM((2 * NVAR * L,), jnp.float32),
            pltpu.VMEM((32,), jnp.float32),
            pltpu.VMEM((TW * L + L,), jnp.float32),
            pltpu.VMEM((TW * L + L,), jnp.float32),
            pltpu.VMEM((8 * CH,), jnp.int32),
            pltpu.VMEM((8 * CH,), jnp.int32),
            pltpu.VMEM((R,), jnp.int32),
            pltpu.VMEM((R,), jnp.int32),
            pltpu.VMEM((NOUT2 * L,), jnp.float32),
            pltpu.VMEM((2 * L,), jnp.float32),
            pltpu.SemaphoreType.DMA,
            pltpu.SemaphoreType.DMA,
        ],
        compiler_params=pltpu.CompilerParams(needs_layout_passes=False),
    )(_sc_body)
    flat = run(xw, csp, rulesf, ocp)
    return flat.reshape(NW, 2, L).transpose(0, 2, 1).reshape(B, 2)
